# baseline (speedup 1.0000x reference)
import functools

import jax
import jax.numpy as jnp
from jax import lax
from jax.experimental import pallas as pl
from jax.experimental.pallas import tpu as pltpu
from jax.experimental.pallas import tpu_sc as plsc

N = 10000
E = 320000
F = 128
G = 16
MAXD = 10
KST = MAXD + 2
KW = 208
NPAD = 10240
NC, NS = 2, 16
NW = NC * NS
CH = 128
KCH = -(-E // (NW * CH * 2)) * 2
EP = NW * CH * KCH
RPS = NPAD // NS
ZR = 32
BN = 1024
GRID = NPAD // BN


def _mesh():
    return plsc.VectorSubcoreMesh(core_axis_name="c", subcore_axis_name="s",
                                  num_cores=NC, num_subcores=NS)


DW = 128


def _sc_deg_body(dst_hbm, ones_hbm, zero_hbm, out_hbm, didx, ones_v, acc_sh):
    c = lax.axis_index("c")
    s = lax.axis_index("s")
    wid = c * NS + s
    pltpu.sync_copy(dst_hbm.at[wid], didx)
    pltpu.sync_copy(ones_hbm, ones_v)

    def zr(t_, carry):
        pltpu.sync_copy(zero_hbm, acc_sh.at[pl.ds(s * RPS + t_ * ZR, ZR)])
        return carry
    lax.fori_loop(0, RPS // ZR, zr, 0)
    plsc.subcore_barrier()

    def chunk(j, carry):
        pltpu.sync_copy(ones_v, acc_sh.at[didx.at[j]], add=True)
        return carry

    lax.fori_loop(0, KCH, chunk, 0)
    plsc.subcore_barrier()
    pltpu.sync_copy(acc_sh.at[pl.ds(s * RPS, RPS)],
                    out_hbm.at[c].at[pl.ds(s * RPS, RPS)])


def _sc_deg(dstp):
    ones = jnp.ones((CH, DW), jnp.float32)
    zero = jnp.zeros((ZR, DW), jnp.float32)
    f = pl.kernel(
        _sc_deg_body,
        out_type=jax.ShapeDtypeStruct((NC, NPAD, DW), jnp.float32),
        mesh=_mesh(),
        scratch_types=[
            pltpu.VMEM((KCH, CH), jnp.int32),
            pltpu.VMEM((CH, DW), jnp.float32),
            pltpu.VMEM_SHARED((NPAD, DW), jnp.float32),
        ],
    )
    return f(dstp, ones, zero)


def _sc_pass_body(src_hbm, dst_hbm, feat_hbm, zero_hbm, out_hbm,
                  didx, sbuf, rows0, rows1, acc_sh, sem0, sem1, semi):
    c = lax.axis_index("c")
    s = lax.axis_index("s")
    wid = c * NS + s
    pltpu.sync_copy(dst_hbm.at[wid], didx)

    def zr(t_, carry):
        pltpu.sync_copy(zero_hbm, acc_sh.at[pl.ds(s * RPS + t_ * ZR, ZR)])
        return carry
    lax.fori_loop(0, RPS // ZR, zr, 0)
    plsc.subcore_barrier()

    def idx_fire(row, slot):
        pltpu.async_copy(src_hbm.at[wid].at[row], sbuf.at[slot], semi)

    def idx_wait(row, slot):
        pltpu.make_async_copy(src_hbm.at[wid].at[row], sbuf.at[slot], semi
                              ).wait()

    for k in range(4):
        idx_fire(k, k)
    idx_wait(0, 0)
    idx_wait(1, 1)
    pltpu.async_copy(feat_hbm.at[sbuf.at[0]], rows0, sem0)
    pltpu.async_copy(feat_hbm.at[sbuf.at[1]], rows1, sem1)

    def pair(jj, carry):
        j = jj * 2
        jn = jnp.minimum(j + 2, KCH - 1)
        jm = jnp.minimum(j + 3, KCH - 1)
        s0 = (jj % 2) * 2

        pltpu.make_async_copy(feat_hbm.at[sbuf.at[s0]], rows0, sem0).wait()
        idx_fire(jnp.minimum(j + 4, KCH - 1), s0)
        pltpu.sync_copy(rows0, acc_sh.at[didx.at[j]], add=True)
        idx_wait(jn, 2 - s0)
        pltpu.async_copy(feat_hbm.at[sbuf.at[2 - s0]], rows0, sem0)

        pltpu.make_async_copy(feat_hbm.at[sbuf.at[s0 + 1]], rows1, sem1).wait()
        idx_fire(jnp.minimum(j + 5, KCH - 1), s0 + 1)
        pltpu.sync_copy(rows1, acc_sh.at[didx.at[j + 1]], add=True)
        idx_wait(jm, 3 - s0)
        pltpu.async_copy(feat_hbm.at[sbuf.at[3 - s0]], rows1, sem1)
        return carry

    lax.fori_loop(0, KCH // 2, pair, 0)
    pltpu.make_async_copy(feat_hbm.at[sbuf.at[0]], rows0, sem0).wait()
    pltpu.make_async_copy(feat_hbm.at[sbuf.at[1]], rows1, sem1).wait()
    idx_wait(KCH - 1, 0)
    idx_wait(KCH - 1, 1)
    plsc.subcore_barrier()
    pltpu.sync_copy(acc_sh.at[pl.ds(s * RPS, RPS)],
                    out_hbm.at[c].at[pl.ds(s * RPS, RPS)])


def _sc_pass(srcp, dstp, feat, zero640):
    f = pl.kernel(
        _sc_pass_body,
        out_type=jax.ShapeDtypeStruct((NC, NPAD, F), jnp.float32),
        mesh=_mesh(),
        scratch_types=[
            pltpu.VMEM((KCH, CH), jnp.int32),
            pltpu.VMEM((4, CH), jnp.int32),
            pltpu.VMEM((CH, F), jnp.float32),
            pltpu.VMEM((CH, F), jnp.float32),
            pltpu.VMEM_SHARED((NPAD, F), jnp.float32),
            pltpu.SemaphoreType.DMA,
            pltpu.SemaphoreType.DMA,
            pltpu.SemaphoreType.DMA,
        ],
    )
    return f(srcp, dstp, feat, zero640)


def _tcA_body(d0, d1, x, bt, xs1, dinv, key):
    deg = d0[:, :1] + d1[:, :1]
    din = lax.rsqrt(deg + 1.0)
    dinv[...] = din
    xs1[...] = x[...] * din
    dc = jnp.minimum(deg.astype(jnp.int32), MAXD)
    key[...] = bt[...] * KST + dc


def _tcA(degp, xp, btp):
    row1 = pl.BlockSpec((BN, 1), lambda i: (i, 0))
    rowD = pl.BlockSpec((BN, DW), lambda i: (i, 0))
    rowF = pl.BlockSpec((BN, F), lambda i: (i, 0))
    return pl.pallas_call(
        _tcA_body,
        grid=(GRID,),
        in_specs=[rowD, rowD, rowF, row1],
        out_specs=[rowF, row1, row1],
        out_shape=[jax.ShapeDtypeStruct((NPAD, F), jnp.float32),
                   jax.ShapeDtypeStruct((NPAD, 1), jnp.float32),
                   jax.ShapeDtypeStruct((NPAD, 1), jnp.int32)],
    )(degp[0], degp[1], xp, btp)


def _tcBC_body(a0, a1, xs, dinv, W, brow, h, xsn):
    p = dinv[...] * (a0[...] + a1[...] + xs[...])
    hv = jnp.dot(p, W[...], preferred_element_type=jnp.float32) + brow[0:1, :]
    h[...] = hv
    xsn[...] = dinv[...] * hv


def _tcBC(acc, xs, dinv, W, b):
    brow = jnp.zeros((8, F), jnp.float32).at[0].set(b)
    row1 = pl.BlockSpec((BN, 1), lambda i: (i, 0))
    rowF = pl.BlockSpec((BN, F), lambda i: (i, 0))
    full = lambda shape: pl.BlockSpec(shape, lambda i: tuple(0 for _ in shape))
    return pl.pallas_call(
        _tcBC_body,
        grid=(GRID,),
        in_specs=[rowF, rowF, rowF, row1, full((F, F)), full((8, F))],
        out_specs=[rowF, rowF],
        out_shape=[jax.ShapeDtypeStruct((NPAD, F), jnp.float32),
                   jax.ShapeDtypeStruct((NPAD, F), jnp.float32)],
    )(acc[0], acc[1], xs, dinv, W, brow)


def _tcD1_body(a0, a1, xs3, ah0, ah1, h2, dinv, bt, ky, P, T, S, cnt):
    i = pl.program_id(0)

    @pl.when(i == 0)
    def _():
        P[...] = jnp.zeros_like(P)
        T[...] = jnp.zeros_like(T)
        S[...] = jnp.zeros_like(S)
        cnt[...] = jnp.zeros_like(cnt)

    polnode = dinv[...] * (a0[...] + a1[...] + xs3[...])
    acch = ah0[...] + ah1[...]
    ob = (bt[...] == lax.broadcasted_iota(jnp.int32, (BN, G), 1)
          ).astype(jnp.float32)
    ok = (ky[...] == lax.broadcasted_iota(jnp.int32, (BN, KW), 1)
          ).astype(jnp.float32)
    dn = (((0,), (0,)), ((), ()))
    P[...] += lax.dot_general(ob, polnode, dn,
                              preferred_element_type=jnp.float32)
    T[...] += lax.dot_general(ok, h2[...], dn,
                              preferred_element_type=jnp.float32)
    S[...] += lax.dot_general(ok, acch, dn,
                              preferred_element_type=jnp.float32)
    cnt[...] += lax.dot_general(ok, jnp.ones((BN, 1), jnp.float32), dn,
                                preferred_element_type=jnp.float32)


def _tcD1(acc3, xs3, acch, h2, dinv, btp, key):
    row1 = pl.BlockSpec((BN, 1), lambda i: (i, 0))
    rowF = pl.BlockSpec((BN, F), lambda i: (i, 0))
    full = lambda shape: pl.BlockSpec(shape, lambda i: tuple(0 for _ in shape))
    return pl.pallas_call(
        _tcD1_body,
        grid=(GRID,),
        in_specs=[rowF, rowF, rowF, rowF, rowF, rowF, row1, row1, row1],
        out_specs=[full((G, F)), full((KW, F)), full((KW, F)), full((KW, 1))],
        out_shape=[jax.ShapeDtypeStruct((G, F), jnp.float32),
                   jax.ShapeDtypeStruct((KW, F), jnp.float32),
                   jax.ShapeDtypeStruct((KW, F), jnp.float32),
                   jax.ShapeDtypeStruct((KW, 1), jnp.float32)],
    )(acc3[0], acc3[1], xs3, acch[0], acch[1], h2, dinv, btp, key)


def _tcD2_body(P, T, S, c, wl, wr, bl, Gm, Wp, bp, pol, val):
    rv = jnp.sum(S[...] * wl[...] + T[...] * wr[...], axis=1, keepdims=True)
    rv = rv + c[...] * bl[...]
    val[...] = jnp.dot(Gm[...], rv, preferred_element_type=jnp.float32)
    counts = jnp.dot(Gm[...], c[...], preferred_element_type=jnp.float32)
    pv = jnp.dot(P[...], Wp[...], preferred_element_type=jnp.float32)
    pol[...] = pv / jnp.maximum(counts, 1.0) + bp[0:1, :]


def _tcD2(P, T, S, c, wlrep, wrrep, blrep, Gmat, W_pol, bp):
    full = lambda shape: pl.BlockSpec(shape, lambda: tuple(0 for _ in shape))
    return pl.pallas_call(
        _tcD2_body,
        in_specs=[full((G, F)), full((KW, F)), full((KW, F)), full((KW, 1)),
                  full((KW, F)), full((KW, F)), full((KW, 1)), full((G, KW)),
                  full((F, G)), full((8, G))],
        out_specs=[full((G, G)), full((G, 1))],
        out_shape=[jax.ShapeDtypeStruct((G, G), jnp.float32),
                   jax.ShapeDtypeStruct((G, 1), jnp.float32)],
    )(P, T, S, c, wlrep, wrrep, blrep, Gmat, W_pol, bp)


def kernel(x, edge_index, batch, W_in, b_in, W_1, b_1,
           Wl_val, bl_val, Wr_val, W_pol, b_pol):
    f32 = jnp.float32
    src = edge_index[0]
    dst = edge_index[1]
    pad_e = EP - E
    srcp = jnp.concatenate([src, jnp.full((pad_e,), N, jnp.int32)]
                           ).reshape(NW, KCH, CH)
    dstp = jnp.concatenate([dst, jnp.full((pad_e,), N, jnp.int32)]
                           ).reshape(NW, KCH, CH)
    xp = jnp.zeros((NPAD, F), f32).at[:N].set(x)
    btp = jnp.concatenate([batch.astype(jnp.int32),
                           jnp.full((NPAD - N,), G, jnp.int32)]
                          ).reshape(NPAD, 1)
    zero640 = jnp.zeros((ZR, F), f32)

    degp = _sc_deg(dstp)
    xs1, dinv, key = _tcA(degp, xp, btp)
    acc1 = _sc_pass(srcp, dstp, xs1, zero640)
    _h1, xs2 = _tcBC(acc1, xs1, dinv, W_in, b_in)
    acc2 = _sc_pass(srcp, dstp, xs2, zero640)
    h2, xs3 = _tcBC(acc2, xs2, dinv, W_1, b_1)
    acc3 = _sc_pass(srcp, dstp, xs3, zero640)
    acch = _sc_pass(srcp, dstp, h2, zero640)
    P, T, S, c = _tcD1(acc3, xs3, acch, h2, dinv, btp, key)

    wl = Wl_val[:, :, 0]
    wr = Wr_val[:, :, 0]
    bl = bl_val[:, 0]
    wl12 = jnp.zeros((KST, F), f32).at[:MAXD + 1].set(wl)
    wr12 = jnp.zeros((KST, F), f32).at[:MAXD + 1].set(wr)
    bl12 = jnp.zeros((KST,), f32).at[:MAXD + 1].set(bl)
    wlrep = jnp.zeros((KW, F), f32).at[:G * KST].set(jnp.tile(wl12, (G, 1)))
    wrrep = jnp.zeros((KW, F), f32).at[:G * KST].set(jnp.tile(wr12, (G, 1)))
    blrep = jnp.zeros((KW, 1), f32).at[:G * KST, 0].set(jnp.tile(bl12, G))
    col = jnp.arange(KW, dtype=jnp.int32)
    Gmat = (((col[None, :] // KST) == jnp.arange(G, dtype=jnp.int32)[:, None])
            & (col[None, :] < G * KST)).astype(f32)
    bp = jnp.zeros((8, G), f32).at[0].set(b_pol)

    pol, val = _tcD2(P, T, S, c, wlrep, wrrep, blrep, Gmat, W_pol, bp)
    return (pol, val)


# speedup vs baseline: 1.7951x; 1.7951x over previous
import functools

import jax
import jax.numpy as jnp
from jax import lax
from jax.experimental import pallas as pl
from jax.experimental.pallas import tpu as pltpu
from jax.experimental.pallas import tpu_sc as plsc

N = 10000
E = 320000
F = 128
G = 16
MAXD = 10
KST = MAXD + 2
KW = 208
NPAD = 10240
NC, NS = 2, 16
NW = NC * NS
CH = 128
TR = -(-E // CH)
KU = -(-TR // NW)
KF = 102
KS = -(-(TR - NS * KF) // NS)
KMAX = KF
RPS = NPAD // NS
ZR = 32
BN = 1024
GRID = NPAD // BN


def _mesh():
    return plsc.VectorSubcoreMesh(core_axis_name="c", subcore_axis_name="s",
                                  num_cores=NC, num_subcores=NS)


DW = 128


def _sc_deg_body(dst_hbm, ones_hbm, zero_hbm, out_hbm, didx, ones_v, acc_sh):
    c = lax.axis_index("c")
    s = lax.axis_index("s")
    wid = c * NS + s
    pltpu.sync_copy(dst_hbm.at[wid], didx)
    pltpu.sync_copy(ones_hbm, ones_v)

    def zr(t_, carry):
        pltpu.sync_copy(zero_hbm, acc_sh.at[pl.ds(s * RPS + t_ * ZR, ZR)])
        return carry
    lax.fori_loop(0, RPS // ZR, zr, 0)
    plsc.subcore_barrier()

    def chunk(j, carry):
        pltpu.sync_copy(ones_v, acc_sh.at[didx.at[j]], add=True)
        return carry

    lax.fori_loop(0, KU, chunk, 0)
    plsc.subcore_barrier()
    pltpu.sync_copy(acc_sh.at[pl.ds(s * RPS, RPS)],
                    out_hbm.at[c].at[pl.ds(s * RPS, RPS)])


def _sc_deg(dstp):
    ones = jnp.ones((CH, DW), jnp.float32)
    zero = jnp.zeros((ZR, DW), jnp.float32)
    f = pl.kernel(
        _sc_deg_body,
        out_type=jax.ShapeDtypeStruct((NC, NPAD, DW), jnp.float32),
        mesh=_mesh(),
        scratch_types=[
            pltpu.VMEM((KU, CH), jnp.int32),
            pltpu.VMEM((CH, DW), jnp.float32),
            pltpu.VMEM_SHARED((NPAD, DW), jnp.float32),
        ],
    )
    return f(dstp, ones, zero)


def _sc_pass_body(src_hbm, dst_hbm, feat_hbm, zero_hbm, out_hbm,
                  sidx, didx, rows, acc_sh, sem):
    c = lax.axis_index("c")
    s = lax.axis_index("s")
    wid = c * NS + s
    kc = jnp.where(c == 0, KF, KS)
    pltpu.sync_copy(src_hbm.at[wid], sidx)
    pltpu.sync_copy(dst_hbm.at[wid], didx)

    def zr(t_, carry):
        pltpu.sync_copy(zero_hbm, acc_sh.at[pl.ds(s * RPS + t_ * ZR, ZR)])
        return carry
    lax.fori_loop(0, RPS // ZR, zr, 0)
    plsc.subcore_barrier()

    def chunk(j, carry):
        pltpu.async_copy(feat_hbm.at[sidx.at[j]], rows, sem).wait()
        pltpu.sync_copy(rows, acc_sh.at[didx.at[j]], add=True)
        return carry

    lax.fori_loop(0, kc, chunk, 0)
    plsc.subcore_barrier()
    pltpu.sync_copy(acc_sh.at[pl.ds(s * RPS, RPS)],
                    out_hbm.at[c].at[pl.ds(s * RPS, RPS)])


def _sc_pass(srcp, dstp, feat, zero640):
    f = pl.kernel(
        _sc_pass_body,
        out_type=jax.ShapeDtypeStruct((NC, NPAD, F), jnp.float32),
        mesh=_mesh(),
        scratch_types=[
            pltpu.VMEM((KMAX, CH), jnp.int32),
            pltpu.VMEM((KMAX, CH), jnp.int32),
            pltpu.VMEM((CH, F), jnp.float32),
            pltpu.VMEM_SHARED((NPAD, F), jnp.float32),
            pltpu.SemaphoreType.DMA,
        ],
    )
    return f(srcp, dstp, feat, zero640)


def _tcA_body(d0, d1, x, bt, xs1, dinv, key):
    deg = d0[:, :1] + d1[:, :1]
    din = lax.rsqrt(deg + 1.0)
    dinv[...] = din
    xs1[...] = x[...] * din
    dc = jnp.minimum(deg.astype(jnp.int32), MAXD)
    key[...] = bt[...] * KST + dc


def _tcA(degp, xp, btp):
    row1 = pl.BlockSpec((BN, 1), lambda i: (i, 0))
    rowD = pl.BlockSpec((BN, DW), lambda i: (i, 0))
    rowF = pl.BlockSpec((BN, F), lambda i: (i, 0))
    return pl.pallas_call(
        _tcA_body,
        grid=(GRID,),
        in_specs=[rowD, rowD, rowF, row1],
        out_specs=[rowF, row1, row1],
        out_shape=[jax.ShapeDtypeStruct((NPAD, F), jnp.float32),
                   jax.ShapeDtypeStruct((NPAD, 1), jnp.float32),
                   jax.ShapeDtypeStruct((NPAD, 1), jnp.int32)],
    )(degp[0], degp[1], xp, btp)


def _tcBC_body(a0, a1, xs, dinv, W, brow, h, xsn):
    p = dinv[...] * (a0[...] + a1[...] + xs[...])
    hv = jnp.dot(p, W[...], preferred_element_type=jnp.float32) + brow[0:1, :]
    h[...] = hv
    xsn[...] = dinv[...] * hv


def _tcBC(acc, xs, dinv, W, b):
    brow = jnp.zeros((8, F), jnp.float32).at[0].set(b)
    row1 = pl.BlockSpec((BN, 1), lambda i: (i, 0))
    rowF = pl.BlockSpec((BN, F), lambda i: (i, 0))
    full = lambda shape: pl.BlockSpec(shape, lambda i: tuple(0 for _ in shape))
    return pl.pallas_call(
        _tcBC_body,
        grid=(GRID,),
        in_specs=[rowF, rowF, rowF, row1, full((F, F)), full((8, F))],
        out_specs=[rowF, rowF],
        out_shape=[jax.ShapeDtypeStruct((NPAD, F), jnp.float32),
                   jax.ShapeDtypeStruct((NPAD, F), jnp.float32)],
    )(acc[0], acc[1], xs, dinv, W, brow)


def _tcD1_body(a0, a1, xs3, ah0, ah1, h2, dinv, bt, ky, P, T, S, cnt):
    i = pl.program_id(0)

    @pl.when(i == 0)
    def _():
        P[...] = jnp.zeros_like(P)
        T[...] = jnp.zeros_like(T)
        S[...] = jnp.zeros_like(S)
        cnt[...] = jnp.zeros_like(cnt)

    polnode = dinv[...] * (a0[...] + a1[...] + xs3[...])
    acch = ah0[...] + ah1[...]
    ob = (bt[...] == lax.broadcasted_iota(jnp.int32, (BN, G), 1)
          ).astype(jnp.float32)
    ok = (ky[...] == lax.broadcasted_iota(jnp.int32, (BN, KW), 1)
          ).astype(jnp.float32)
    dn = (((0,), (0,)), ((), ()))
    P[...] += lax.dot_general(ob, polnode, dn,
                              preferred_element_type=jnp.float32)
    T[...] += lax.dot_general(ok, h2[...], dn,
                              preferred_element_type=jnp.float32)
    S[...] += lax.dot_general(ok, acch, dn,
                              preferred_element_type=jnp.float32)
    cnt[...] += lax.dot_general(ok, jnp.ones((BN, 1), jnp.float32), dn,
                                preferred_element_type=jnp.float32)


def _tcD1(acc3, xs3, acch, h2, dinv, btp, key):
    row1 = pl.BlockSpec((BN, 1), lambda i: (i, 0))
    rowF = pl.BlockSpec((BN, F), lambda i: (i, 0))
    full = lambda shape: pl.BlockSpec(shape, lambda i: tuple(0 for _ in shape))
    return pl.pallas_call(
        _tcD1_body,
        grid=(GRID,),
        in_specs=[rowF, rowF, rowF, rowF, rowF, rowF, row1, row1, row1],
        out_specs=[full((G, F)), full((KW, F)), full((KW, F)), full((KW, 1))],
        out_shape=[jax.ShapeDtypeStruct((G, F), jnp.float32),
                   jax.ShapeDtypeStruct((KW, F), jnp.float32),
                   jax.ShapeDtypeStruct((KW, F), jnp.float32),
                   jax.ShapeDtypeStruct((KW, 1), jnp.float32)],
    )(acc3[0], acc3[1], xs3, acch[0], acch[1], h2, dinv, btp, key)


def _tcD2_body(P, T, S, c, wl, wr, bl, Gm, Wp, bp, pol, val):
    rv = jnp.sum(S[...] * wl[...] + T[...] * wr[...], axis=1, keepdims=True)
    rv = rv + c[...] * bl[...]
    val[...] = jnp.dot(Gm[...], rv, preferred_element_type=jnp.float32)
    counts = jnp.dot(Gm[...], c[...], preferred_element_type=jnp.float32)
    pv = jnp.dot(P[...], Wp[...], preferred_element_type=jnp.float32)
    pol[...] = pv / jnp.maximum(counts, 1.0) + bp[0:1, :]


def _tcD2(P, T, S, c, wlrep, wrrep, blrep, Gmat, W_pol, bp):
    full = lambda shape: pl.BlockSpec(shape, lambda: tuple(0 for _ in shape))
    return pl.pallas_call(
        _tcD2_body,
        in_specs=[full((G, F)), full((KW, F)), full((KW, F)), full((KW, 1)),
                  full((KW, F)), full((KW, F)), full((KW, 1)), full((G, KW)),
                  full((F, G)), full((8, G))],
        out_specs=[full((G, G)), full((G, 1))],
        out_shape=[jax.ShapeDtypeStruct((G, G), jnp.float32),
                   jax.ShapeDtypeStruct((G, 1), jnp.float32)],
    )(P, T, S, c, wlrep, wrrep, blrep, Gmat, W_pol, bp)


def kernel(x, edge_index, batch, W_in, b_in, W_1, b_1,
           Wl_val, bl_val, Wr_val, W_pol, b_pol):
    f32 = jnp.float32
    src = edge_index[0]
    dst = edge_index[1]
    def chunk_rows(ix):
        r = jnp.concatenate([ix, jnp.full((TR * CH - E,), N, jnp.int32)]
                            ).reshape(TR, CH)
        return r

    def skewed(ix):
        r = chunk_rows(ix)
        padrow = jnp.full((NS * (KF + KS) - TR, CH), N, jnp.int32)
        r = jnp.concatenate([r, padrow])
        p0 = r[:NS * KF].reshape(NS, KF, CH)
        p1 = r[NS * KF:].reshape(NS, KS, CH)
        p1 = jnp.concatenate(
            [p1, jnp.full((NS, KMAX - KS, CH), N, jnp.int32)], axis=1)
        return jnp.concatenate([p0, p1])

    def uniform(ix):
        r = chunk_rows(ix)
        r = jnp.concatenate(
            [r, jnp.full((NW * KU - TR, CH), N, jnp.int32)])
        return r.reshape(NW, KU, CH)

    srcp = skewed(src)
    dstp = skewed(dst)
    dstu = uniform(dst)
    xp = jnp.zeros((NPAD, F), f32).at[:N].set(x)
    btp = jnp.concatenate([batch.astype(jnp.int32),
                           jnp.full((NPAD - N,), G, jnp.int32)]
                          ).reshape(NPAD, 1)
    zero640 = jnp.zeros((ZR, F), f32)

    degp = _sc_deg(dstu)
    xs1, dinv, key = _tcA(degp, xp, btp)
    acc1 = _sc_pass(srcp, dstp, xs1, zero640)
    _h1, xs2 = _tcBC(acc1, xs1, dinv, W_in, b_in)
    acc2 = _sc_pass(srcp, dstp, xs2, zero640)
    h2, xs3 = _tcBC(acc2, xs2, dinv, W_1, b_1)
    acc3 = _sc_pass(srcp, dstp, xs3, zero640)
    acch = _sc_pass(srcp, dstp, h2, zero640)
    P, T, S, c = _tcD1(acc3, xs3, acch, h2, dinv, btp, key)

    wl = Wl_val[:, :, 0]
    wr = Wr_val[:, :, 0]
    bl = bl_val[:, 0]
    wl12 = jnp.zeros((KST, F), f32).at[:MAXD + 1].set(wl)
    wr12 = jnp.zeros((KST, F), f32).at[:MAXD + 1].set(wr)
    bl12 = jnp.zeros((KST,), f32).at[:MAXD + 1].set(bl)
    wlrep = jnp.zeros((KW, F), f32).at[:G * KST].set(jnp.tile(wl12, (G, 1)))
    wrrep = jnp.zeros((KW, F), f32).at[:G * KST].set(jnp.tile(wr12, (G, 1)))
    blrep = jnp.zeros((KW, 1), f32).at[:G * KST, 0].set(jnp.tile(bl12, G))
    col = jnp.arange(KW, dtype=jnp.int32)
    Gmat = (((col[None, :] // KST) == jnp.arange(G, dtype=jnp.int32)[:, None])
            & (col[None, :] < G * KST)).astype(f32)
    bp = jnp.zeros((8, G), f32).at[0].set(b_pol)

    pol, val = _tcD2(P, T, S, c, wlrep, wrrep, blrep, Gmat, W_pol, bp)
    return (pol, val)


# speedup vs baseline: 1.9776x; 1.1016x over previous
import functools

import jax
import jax.numpy as jnp
from jax import lax
from jax.experimental import pallas as pl
from jax.experimental.pallas import tpu as pltpu
from jax.experimental.pallas import tpu_sc as plsc

N = 10000
E = 320000
F = 128
G = 16
MAXD = 10
KST = MAXD + 2
KW = 208
NPAD = 10240
NC, NS = 2, 16
NW = NC * NS
CH = 128
TR = -(-E // CH)
KU = -(-TR // NW)
KF = 98
KS = -(-(TR - NS * KF) // NS)
KMAX = KF
RPS = NPAD // NS
ZR = 32
BN = 1024
GRID = NPAD // BN


def _mesh():
    return plsc.VectorSubcoreMesh(core_axis_name="c", subcore_axis_name="s",
                                  num_cores=NC, num_subcores=NS)


def _sc_deg_body(dst_hbm, one_hbm, out_hbm, didx, hist, blk, ones_v, hist_sh):
    c = lax.axis_index("c")
    s = lax.axis_index("s")
    wid = c * NS + s
    pltpu.sync_copy(dst_hbm.at[wid], didx)
    pltpu.sync_copy(one_hbm, ones_v)

    ones16 = ones_v[...]
    zeros16 = ones16 - ones16

    def z(i, carry):
        hist[pl.ds(i * 16, 16)] = zeros16
        return carry
    lax.fori_loop(0, NPAD // 16, z, 0)

    def chunk(j, carry):
        for t_ in range(CH // 16):
            dv = didx[j, pl.ds(t_ * 16, 16)]
            plsc.addupdate_scatter(hist, [dv], ones16)
        return carry
    lax.fori_loop(0, KU, chunk, 0)

    pltpu.sync_copy(hist, hist_sh.at[s])
    plsc.subcore_barrier()
    for t_ in range(NS):
        pltpu.sync_copy(hist_sh.at[t_].at[pl.ds(s * RPS, RPS)], blk.at[t_])

    def red(pp, carry):
        acc = blk[0, pl.ds(pp * 16, 16)]
        for t_ in range(1, NS):
            acc = acc + blk[t_, pl.ds(pp * 16, 16)]
        hist[pl.ds(pp * 16, 16)] = acc
        return carry
    lax.fori_loop(0, RPS // 16, red, 0)
    pltpu.sync_copy(hist.at[pl.ds(0, RPS)],
                    out_hbm.at[c].at[pl.ds(s * RPS, RPS)])


def _sc_deg(dstu):
    f = pl.kernel(
        _sc_deg_body,
        out_type=jax.ShapeDtypeStruct((NC, NPAD), jnp.float32),
        mesh=_mesh(),
        scratch_types=[
            pltpu.VMEM((KU, CH), jnp.int32),
            pltpu.VMEM((NPAD,), jnp.float32),
            pltpu.VMEM((NS, RPS), jnp.float32),
            pltpu.VMEM((16,), jnp.float32),
            pltpu.VMEM_SHARED((NS, NPAD), jnp.float32),
        ],
        compiler_params=pltpu.CompilerParams(needs_layout_passes=False),
    )
    return f(dstu, jnp.ones((16,), jnp.float32))


def _sc_pass_body(src_hbm, dst_hbm, feat_hbm, zero_hbm, out_hbm,
                  sidx, didx, rows, acc_sh, sem):
    c = lax.axis_index("c")
    s = lax.axis_index("s")
    wid = c * NS + s
    kc = jnp.where(c == 0, KF, KS)
    pltpu.sync_copy(src_hbm.at[wid], sidx)
    pltpu.sync_copy(dst_hbm.at[wid], didx)

    def zr(t_, carry):
        pltpu.sync_copy(zero_hbm, acc_sh.at[pl.ds(s * RPS + t_ * ZR, ZR)])
        return carry
    lax.fori_loop(0, RPS // ZR, zr, 0)
    plsc.subcore_barrier()

    def chunk(j, carry):
        pltpu.async_copy(feat_hbm.at[sidx.at[j]], rows, sem).wait()
        pltpu.sync_copy(rows, acc_sh.at[didx.at[j]], add=True)
        return carry

    lax.fori_loop(0, kc, chunk, 0)
    plsc.subcore_barrier()
    pltpu.sync_copy(acc_sh.at[pl.ds(s * RPS, RPS)],
                    out_hbm.at[c].at[pl.ds(s * RPS, RPS)])


def _sc_pass(srcp, dstp, feat, zero640):
    f = pl.kernel(
        _sc_pass_body,
        out_type=jax.ShapeDtypeStruct((NC, NPAD, F), jnp.float32),
        mesh=_mesh(),
        scratch_types=[
            pltpu.VMEM((KMAX, CH), jnp.int32),
            pltpu.VMEM((KMAX, CH), jnp.int32),
            pltpu.VMEM((CH, F), jnp.float32),
            pltpu.VMEM_SHARED((NPAD, F), jnp.float32),
            pltpu.SemaphoreType.DMA,
        ],
    )
    return f(srcp, dstp, feat, zero640)


def _tcA_body(d0, d1, x, bt, xs1, dinv, key):
    deg = d0[...] + d1[...]
    din = lax.rsqrt(deg + 1.0)
    dinv[...] = din
    xs1[...] = x[...] * din
    dc = jnp.minimum(deg.astype(jnp.int32), MAXD)
    key[...] = bt[...] * KST + dc


def _tcA(degp, xp, btp):
    row1 = pl.BlockSpec((BN, 1), lambda i: (i, 0))
    rowF = pl.BlockSpec((BN, F), lambda i: (i, 0))
    return pl.pallas_call(
        _tcA_body,
        grid=(GRID,),
        in_specs=[row1, row1, rowF, row1],
        out_specs=[rowF, row1, row1],
        out_shape=[jax.ShapeDtypeStruct((NPAD, F), jnp.float32),
                   jax.ShapeDtypeStruct((NPAD, 1), jnp.float32),
                   jax.ShapeDtypeStruct((NPAD, 1), jnp.int32)],
    )(degp[0].reshape(NPAD, 1), degp[1].reshape(NPAD, 1), xp, btp)


def _tcBC_body(a0, a1, xs, dinv, W, brow, h, xsn):
    p = dinv[...] * (a0[...] + a1[...] + xs[...])
    hv = jnp.dot(p, W[...], preferred_element_type=jnp.float32) + brow[0:1, :]
    h[...] = hv
    xsn[...] = dinv[...] * hv


def _tcBC(acc, xs, dinv, W, b):
    brow = jnp.zeros((8, F), jnp.float32).at[0].set(b)
    row1 = pl.BlockSpec((BN, 1), lambda i: (i, 0))
    rowF = pl.BlockSpec((BN, F), lambda i: (i, 0))
    full = lambda shape: pl.BlockSpec(shape, lambda i: tuple(0 for _ in shape))
    return pl.pallas_call(
        _tcBC_body,
        grid=(GRID,),
        in_specs=[rowF, rowF, rowF, row1, full((F, F)), full((8, F))],
        out_specs=[rowF, rowF],
        out_shape=[jax.ShapeDtypeStruct((NPAD, F), jnp.float32),
                   jax.ShapeDtypeStruct((NPAD, F), jnp.float32)],
    )(acc[0], acc[1], xs, dinv, W, brow)


def _tcD1_body(a0, a1, xs3, ah0, ah1, h2, dinv, bt, ky, P, T, S, cnt):
    i = pl.program_id(0)

    @pl.when(i == 0)
    def _():
        P[...] = jnp.zeros_like(P)
        T[...] = jnp.zeros_like(T)
        S[...] = jnp.zeros_like(S)
        cnt[...] = jnp.zeros_like(cnt)

    polnode = dinv[...] * (a0[...] + a1[...] + xs3[...])
    acch = ah0[...] + ah1[...]
    ob = (bt[...] == lax.broadcasted_iota(jnp.int32, (BN, G), 1)
          ).astype(jnp.float32)
    ok = (ky[...] == lax.broadcasted_iota(jnp.int32, (BN, KW), 1)
          ).astype(jnp.float32)
    dn = (((0,), (0,)), ((), ()))
    P[...] += lax.dot_general(ob, polnode, dn,
                              preferred_element_type=jnp.float32)
    T[...] += lax.dot_general(ok, h2[...], dn,
                              preferred_element_type=jnp.float32)
    S[...] += lax.dot_general(ok, acch, dn,
                              preferred_element_type=jnp.float32)
    cnt[...] += lax.dot_general(ok, jnp.ones((BN, 1), jnp.float32), dn,
                                preferred_element_type=jnp.float32)


def _tcD1(acc3, xs3, acch, h2, dinv, btp, key):
    row1 = pl.BlockSpec((BN, 1), lambda i: (i, 0))
    rowF = pl.BlockSpec((BN, F), lambda i: (i, 0))
    full = lambda shape: pl.BlockSpec(shape, lambda i: tuple(0 for _ in shape))
    return pl.pallas_call(
        _tcD1_body,
        grid=(GRID,),
        in_specs=[rowF, rowF, rowF, rowF, rowF, rowF, row1, row1, row1],
        out_specs=[full((G, F)), full((KW, F)), full((KW, F)), full((KW, 1))],
        out_shape=[jax.ShapeDtypeStruct((G, F), jnp.float32),
                   jax.ShapeDtypeStruct((KW, F), jnp.float32),
                   jax.ShapeDtypeStruct((KW, F), jnp.float32),
                   jax.ShapeDtypeStruct((KW, 1), jnp.float32)],
    )(acc3[0], acc3[1], xs3, acch[0], acch[1], h2, dinv, btp, key)


def _tcD2_body(P, T, S, c, wl, wr, bl, Gm, Wp, bp, pol, val):
    rv = jnp.sum(S[...] * wl[...] + T[...] * wr[...], axis=1, keepdims=True)
    rv = rv + c[...] * bl[...]
    val[...] = jnp.dot(Gm[...], rv, preferred_element_type=jnp.float32)
    counts = jnp.dot(Gm[...], c[...], preferred_element_type=jnp.float32)
    pv = jnp.dot(P[...], Wp[...], preferred_element_type=jnp.float32)
    pol[...] = pv / jnp.maximum(counts, 1.0) + bp[0:1, :]


def _tcD2(P, T, S, c, wlrep, wrrep, blrep, Gmat, W_pol, bp):
    full = lambda shape: pl.BlockSpec(shape, lambda: tuple(0 for _ in shape))
    return pl.pallas_call(
        _tcD2_body,
        in_specs=[full((G, F)), full((KW, F)), full((KW, F)), full((KW, 1)),
                  full((KW, F)), full((KW, F)), full((KW, 1)), full((G, KW)),
                  full((F, G)), full((8, G))],
        out_specs=[full((G, G)), full((G, 1))],
        out_shape=[jax.ShapeDtypeStruct((G, G), jnp.float32),
                   jax.ShapeDtypeStruct((G, 1), jnp.float32)],
    )(P, T, S, c, wlrep, wrrep, blrep, Gmat, W_pol, bp)


def kernel(x, edge_index, batch, W_in, b_in, W_1, b_1,
           Wl_val, bl_val, Wr_val, W_pol, b_pol):
    f32 = jnp.float32
    src = edge_index[0]
    dst = edge_index[1]
    def chunk_rows(ix):
        r = jnp.concatenate([ix, jnp.full((TR * CH - E,), N, jnp.int32)]
                            ).reshape(TR, CH)
        return r

    def skewed(ix):
        r = chunk_rows(ix)
        padrow = jnp.full((NS * (KF + KS) - TR, CH), N, jnp.int32)
        r = jnp.concatenate([r, padrow])
        p0 = r[:NS * KF].reshape(NS, KF, CH)
        p1 = r[NS * KF:].reshape(NS, KS, CH)
        p1 = jnp.concatenate(
            [p1, jnp.full((NS, KMAX - KS, CH), N, jnp.int32)], axis=1)
        return jnp.concatenate([p0, p1])

    def uniform(ix):
        r = chunk_rows(ix)
        r = jnp.concatenate(
            [r, jnp.full((NW * KU - TR, CH), N, jnp.int32)])
        return r.reshape(NW, KU, CH)

    srcp = skewed(src)
    dstp = skewed(dst)
    dstu = uniform(dst)
    xp = jnp.zeros((NPAD, F), f32).at[:N].set(x)
    btp = jnp.concatenate([batch.astype(jnp.int32),
                           jnp.full((NPAD - N,), G, jnp.int32)]
                          ).reshape(NPAD, 1)
    zero640 = jnp.zeros((ZR, F), f32)

    degp = _sc_deg(dstu)
    xs1, dinv, key = _tcA(degp, xp, btp)
    acc1 = _sc_pass(srcp, dstp, xs1, zero640)
    _h1, xs2 = _tcBC(acc1, xs1, dinv, W_in, b_in)
    acc2 = _sc_pass(srcp, dstp, xs2, zero640)
    h2, xs3 = _tcBC(acc2, xs2, dinv, W_1, b_1)
    acc3 = _sc_pass(srcp, dstp, xs3, zero640)
    acch = _sc_pass(srcp, dstp, h2, zero640)
    P, T, S, c = _tcD1(acc3, xs3, acch, h2, dinv, btp, key)

    wl = Wl_val[:, :, 0]
    wr = Wr_val[:, :, 0]
    bl = bl_val[:, 0]
    wl12 = jnp.zeros((KST, F), f32).at[:MAXD + 1].set(wl)
    wr12 = jnp.zeros((KST, F), f32).at[:MAXD + 1].set(wr)
    bl12 = jnp.zeros((KST,), f32).at[:MAXD + 1].set(bl)
    wlrep = jnp.zeros((KW, F), f32).at[:G * KST].set(jnp.tile(wl12, (G, 1)))
    wrrep = jnp.zeros((KW, F), f32).at[:G * KST].set(jnp.tile(wr12, (G, 1)))
    blrep = jnp.zeros((KW, 1), f32).at[:G * KST, 0].set(jnp.tile(bl12, G))
    col = jnp.arange(KW, dtype=jnp.int32)
    Gmat = (((col[None, :] // KST) == jnp.arange(G, dtype=jnp.int32)[:, None])
            & (col[None, :] < G * KST)).astype(f32)
    bp = jnp.zeros((8, G), f32).at[0].set(b_pol)

    pol, val = _tcD2(P, T, S, c, wlrep, wrrep, blrep, Gmat, W_pol, bp)
    return (pol, val)


# speedup vs baseline: 2.1418x; 1.0831x over previous
import functools

import jax
import jax.numpy as jnp
from jax import lax
from jax.experimental import pallas as pl
from jax.experimental.pallas import tpu as pltpu
from jax.experimental.pallas import tpu_sc as plsc

N = 10000
E = 320000
F = 128
G = 16
MAXD = 10
KST = MAXD + 2
KW = 208
NPAD = 10240
NC, NS = 2, 16
NW = NC * NS
CH = 128
TR = -(-E // CH)
KF = 98
KS = -(-(TR - NS * KF) // NS)
KMAX = KF
RPS = NPAD // NS
ZR = 320
BN = 1024
GRID = NPAD // BN


def _mesh():
    return plsc.VectorSubcoreMesh(core_axis_name="c", subcore_axis_name="s",
                                  num_cores=NC, num_subcores=NS)


def _sc_deg_body(dst_hbm, one_hbm, out_hbm, didx, hist, blk, ones_v, hist_sh):
    c = lax.axis_index("c")
    s = lax.axis_index("s")
    wid = c * NS + s
    pltpu.sync_copy(dst_hbm.at[wid], didx)
    pltpu.sync_copy(one_hbm, ones_v)

    ones16 = ones_v[...]
    zeros16 = ones16 - ones16

    def z(i, carry):
        hist[pl.ds(i * 16, 16)] = zeros16
        return carry
    lax.fori_loop(0, NPAD // 16, z, 0)

    kc = jnp.where(c == 0, KF, KS)

    def chunk(j, carry):
        for t_ in range(CH // 16):
            dv = didx[j, pl.ds(t_ * 16, 16)]
            plsc.addupdate_scatter(hist, [dv], ones16)
        return carry
    lax.fori_loop(0, kc, chunk, 0)

    pltpu.sync_copy(hist, hist_sh.at[s])
    plsc.subcore_barrier()
    for t_ in range(NS):
        pltpu.sync_copy(hist_sh.at[t_].at[pl.ds(s * RPS, RPS)], blk.at[t_])

    def red(pp, carry):
        acc = blk[0, pl.ds(pp * 16, 16)]
        for t_ in range(1, NS):
            acc = acc + blk[t_, pl.ds(pp * 16, 16)]
        hist[pl.ds(pp * 16, 16)] = acc
        return carry
    lax.fori_loop(0, RPS // 16, red, 0)
    pltpu.sync_copy(hist.at[pl.ds(0, RPS)],
                    out_hbm.at[c].at[pl.ds(s * RPS, RPS)])


def _sc_deg(dstu):
    f = pl.kernel(
        _sc_deg_body,
        out_type=jax.ShapeDtypeStruct((NC, NPAD), jnp.float32),
        mesh=_mesh(),
        scratch_types=[
            pltpu.VMEM((KMAX, CH), jnp.int32),
            pltpu.VMEM((NPAD,), jnp.float32),
            pltpu.VMEM((NS, RPS), jnp.float32),
            pltpu.VMEM((16,), jnp.float32),
            pltpu.VMEM_SHARED((NS, NPAD), jnp.float32),
        ],
        compiler_params=pltpu.CompilerParams(needs_layout_passes=False),
    )
    return f(dstu, jnp.ones((16,), jnp.float32))


def _sc_pass_body(src_hbm, dst_hbm, feat_hbm, zero_hbm, out_hbm,
                  sidx, didx, rows, acc_sh, sem):
    c = lax.axis_index("c")
    s = lax.axis_index("s")
    wid = c * NS + s
    kc = jnp.where(c == 0, KF, KS)
    pltpu.sync_copy(src_hbm.at[wid], sidx)
    pltpu.sync_copy(dst_hbm.at[wid], didx)

    def zr(t_, carry):
        pltpu.sync_copy(zero_hbm, acc_sh.at[pl.ds(s * RPS + t_ * ZR, ZR)])
        return carry
    lax.fori_loop(0, RPS // ZR, zr, 0)
    plsc.subcore_barrier()

    def chunk(j, carry):
        pltpu.async_copy(feat_hbm.at[sidx.at[j]], rows, sem).wait()
        pltpu.sync_copy(rows, acc_sh.at[didx.at[j]], add=True)
        return carry

    lax.fori_loop(0, kc, chunk, 0)
    plsc.subcore_barrier()
    pltpu.sync_copy(acc_sh.at[pl.ds(s * RPS, RPS)],
                    out_hbm.at[c].at[pl.ds(s * RPS, RPS)])


def _sc_pass(srcp, dstp, feat, zero640):
    f = pl.kernel(
        _sc_pass_body,
        out_type=jax.ShapeDtypeStruct((NC, NPAD, F), jnp.float32),
        mesh=_mesh(),
        scratch_types=[
            pltpu.VMEM((KMAX, CH), jnp.int32),
            pltpu.VMEM((KMAX, CH), jnp.int32),
            pltpu.VMEM((CH, F), jnp.float32),
            pltpu.VMEM_SHARED((NPAD, F), jnp.float32),
            pltpu.SemaphoreType.DMA,
        ],
    )
    return f(srcp, dstp, feat, zero640)


def _tcA_body(d0, d1, x, bt, xs1, dinv, key):
    deg = d0[...] + d1[...]
    din = lax.rsqrt(deg + 1.0)
    dinv[...] = din
    xs1[...] = x[...] * din
    dc = jnp.minimum(deg.astype(jnp.int32), MAXD)
    key[...] = bt[...] * KST + dc


def _tcA(degp, xp, btp):
    row1 = pl.BlockSpec((BN, 1), lambda i: (i, 0))
    rowF = pl.BlockSpec((BN, F), lambda i: (i, 0))
    return pl.pallas_call(
        _tcA_body,
        grid=(GRID,),
        in_specs=[row1, row1, rowF, row1],
        out_specs=[rowF, row1, row1],
        out_shape=[jax.ShapeDtypeStruct((NPAD, F), jnp.float32),
                   jax.ShapeDtypeStruct((NPAD, 1), jnp.float32),
                   jax.ShapeDtypeStruct((NPAD, 1), jnp.int32)],
    )(degp[0].reshape(NPAD, 1), degp[1].reshape(NPAD, 1), xp, btp)


def _tcBC_body(a0, a1, xs, dinv, W, brow, h, xsn):
    p = dinv[...] * (a0[0] + a1[0] + xs[...])
    hv = jnp.dot(p, W[...], preferred_element_type=jnp.float32) + brow[0:1, :]
    h[...] = hv
    xsn[...] = dinv[...] * hv


def _tcBC(acc, xs, dinv, W, b):
    brow = jnp.zeros((8, F), jnp.float32).at[0].set(b)
    row1 = pl.BlockSpec((BN, 1), lambda i: (i, 0))
    rowF = pl.BlockSpec((BN, F), lambda i: (i, 0))
    p0 = pl.BlockSpec((1, BN, F), lambda i: (0, i, 0))
    p1 = pl.BlockSpec((1, BN, F), lambda i: (1, i, 0))
    full = lambda shape: pl.BlockSpec(shape, lambda i: tuple(0 for _ in shape))
    return pl.pallas_call(
        _tcBC_body,
        grid=(GRID,),
        in_specs=[p0, p1, rowF, row1, full((F, F)), full((8, F))],
        out_specs=[rowF, rowF],
        out_shape=[jax.ShapeDtypeStruct((NPAD, F), jnp.float32),
                   jax.ShapeDtypeStruct((NPAD, F), jnp.float32)],
    )(acc, acc, xs, dinv, W, brow)


def _tcD1_body(a0, a1, xs3, ah0, ah1, h2, dinv, bt, ky, P, T, S, cnt):
    i = pl.program_id(0)

    @pl.when(i == 0)
    def _():
        P[...] = jnp.zeros_like(P)
        T[...] = jnp.zeros_like(T)
        S[...] = jnp.zeros_like(S)
        cnt[...] = jnp.zeros_like(cnt)

    polnode = dinv[...] * (a0[0] + a1[0] + xs3[...])
    acch = ah0[0] + ah1[0]
    ob = (bt[...] == lax.broadcasted_iota(jnp.int32, (BN, G), 1)
          ).astype(jnp.float32)
    ok = (ky[...] == lax.broadcasted_iota(jnp.int32, (BN, KW), 1)
          ).astype(jnp.float32)
    dn = (((0,), (0,)), ((), ()))
    P[...] += lax.dot_general(ob, polnode, dn,
                              preferred_element_type=jnp.float32)
    T[...] += lax.dot_general(ok, h2[...], dn,
                              preferred_element_type=jnp.float32)
    S[...] += lax.dot_general(ok, acch, dn,
                              preferred_element_type=jnp.float32)
    cnt[...] += lax.dot_general(ok, jnp.ones((BN, 1), jnp.float32), dn,
                                preferred_element_type=jnp.float32)


def _tcD1(acc3, xs3, acch, h2, dinv, btp, key):
    row1 = pl.BlockSpec((BN, 1), lambda i: (i, 0))
    rowF = pl.BlockSpec((BN, F), lambda i: (i, 0))
    p0 = pl.BlockSpec((1, BN, F), lambda i: (0, i, 0))
    p1 = pl.BlockSpec((1, BN, F), lambda i: (1, i, 0))
    full = lambda shape: pl.BlockSpec(shape, lambda i: tuple(0 for _ in shape))
    return pl.pallas_call(
        _tcD1_body,
        grid=(GRID,),
        in_specs=[p0, p1, rowF, p0, p1, rowF, row1, row1, row1],
        out_specs=[full((G, F)), full((KW, F)), full((KW, F)), full((KW, 1))],
        out_shape=[jax.ShapeDtypeStruct((G, F), jnp.float32),
                   jax.ShapeDtypeStruct((KW, F), jnp.float32),
                   jax.ShapeDtypeStruct((KW, F), jnp.float32),
                   jax.ShapeDtypeStruct((KW, 1), jnp.float32)],
    )(acc3, acc3, xs3, acch, acch, h2, dinv, btp, key)


def _tcD2_body(P, T, S, c, wl, wr, bl, Gm, Wp, bp, pol, val):
    rv = jnp.sum(S[...] * wl[...] + T[...] * wr[...], axis=1, keepdims=True)
    rv = rv + c[...] * bl[...]
    val[...] = jnp.dot(Gm[...], rv, preferred_element_type=jnp.float32)
    counts = jnp.dot(Gm[...], c[...], preferred_element_type=jnp.float32)
    pv = jnp.dot(P[...], Wp[...], preferred_element_type=jnp.float32)
    pol[...] = pv / jnp.maximum(counts, 1.0) + bp[0:1, :]


def _tcD2(P, T, S, c, wlrep, wrrep, blrep, Gmat, W_pol, bp):
    full = lambda shape: pl.BlockSpec(shape, lambda: tuple(0 for _ in shape))
    return pl.pallas_call(
        _tcD2_body,
        in_specs=[full((G, F)), full((KW, F)), full((KW, F)), full((KW, 1)),
                  full((KW, F)), full((KW, F)), full((KW, 1)), full((G, KW)),
                  full((F, G)), full((8, G))],
        out_specs=[full((G, G)), full((G, 1))],
        out_shape=[jax.ShapeDtypeStruct((G, G), jnp.float32),
                   jax.ShapeDtypeStruct((G, 1), jnp.float32)],
    )(P, T, S, c, wlrep, wrrep, blrep, Gmat, W_pol, bp)


def kernel(x, edge_index, batch, W_in, b_in, W_1, b_1,
           Wl_val, bl_val, Wr_val, W_pol, b_pol):
    f32 = jnp.float32
    src = edge_index[0]
    dst = edge_index[1]
    def chunk_rows(ix):
        r = jnp.concatenate([ix, jnp.full((TR * CH - E,), N, jnp.int32)]
                            ).reshape(TR, CH)
        return r

    def skewed(ix):
        r = chunk_rows(ix)
        padrow = jnp.full((NS * (KF + KS) - TR, CH), N, jnp.int32)
        r = jnp.concatenate([r, padrow])
        p0 = r[:NS * KF].reshape(NS, KF, CH)
        p1 = r[NS * KF:].reshape(NS, KS, CH)
        p1 = jnp.concatenate(
            [p1, jnp.full((NS, KMAX - KS, CH), N, jnp.int32)], axis=1)
        return jnp.concatenate([p0, p1])

    srcp = skewed(src)
    dstp = skewed(dst)
    xp = jnp.zeros((NPAD, F), f32).at[:N].set(x)
    btp = jnp.concatenate([batch.astype(jnp.int32),
                           jnp.full((NPAD - N,), G, jnp.int32)]
                          ).reshape(NPAD, 1)
    zero640 = jnp.zeros((ZR, F), f32)

    degp = _sc_deg(dstp)
    xs1, dinv, key = _tcA(degp, xp, btp)
    acc1 = _sc_pass(srcp, dstp, xs1, zero640)
    _h1, xs2 = _tcBC(acc1, xs1, dinv, W_in, b_in)
    acc2 = _sc_pass(srcp, dstp, xs2, zero640)
    h2, xs3 = _tcBC(acc2, xs2, dinv, W_1, b_1)
    acc3 = _sc_pass(srcp, dstp, xs3, zero640)
    acch = _sc_pass(srcp, dstp, h2, zero640)
    P, T, S, c = _tcD1(acc3, xs3, acch, h2, dinv, btp, key)

    wl = Wl_val[:, :, 0]
    wr = Wr_val[:, :, 0]
    bl = bl_val[:, 0]
    wl12 = jnp.zeros((KST, F), f32).at[:MAXD + 1].set(wl)
    wr12 = jnp.zeros((KST, F), f32).at[:MAXD + 1].set(wr)
    bl12 = jnp.zeros((KST,), f32).at[:MAXD + 1].set(bl)
    wlrep = jnp.zeros((KW, F), f32).at[:G * KST].set(jnp.tile(wl12, (G, 1)))
    wrrep = jnp.zeros((KW, F), f32).at[:G * KST].set(jnp.tile(wr12, (G, 1)))
    blrep = jnp.zeros((KW, 1), f32).at[:G * KST, 0].set(jnp.tile(bl12, G))
    col = jnp.arange(KW, dtype=jnp.int32)
    Gmat = (((col[None, :] // KST) == jnp.arange(G, dtype=jnp.int32)[:, None])
            & (col[None, :] < G * KST)).astype(f32)
    bp = jnp.zeros((8, G), f32).at[0].set(b_pol)

    pol, val = _tcD2(P, T, S, c, wlrep, wrrep, blrep, Gmat, W_pol, bp)
    return (pol, val)


# speedup vs baseline: 2.1574x; 1.0072x over previous
import functools

import jax
import jax.numpy as jnp
from jax import lax
from jax.experimental import pallas as pl
from jax.experimental.pallas import tpu as pltpu
from jax.experimental.pallas import tpu_sc as plsc

N = 10000
E = 320000
F = 128
G = 16
MAXD = 10
KST = MAXD + 2
KW = 208
NPAD = 10240
NC, NS = 2, 16
NW = NC * NS
CH = 128
TR = -(-E // CH)
KF = 96
KS = -(-(TR - NS * KF) // NS)
KMAX = KF
RPS = NPAD // NS
ZR = 320
BN = 1024
GRID = NPAD // BN


def _mesh():
    return plsc.VectorSubcoreMesh(core_axis_name="c", subcore_axis_name="s",
                                  num_cores=NC, num_subcores=NS)


def _sc_deg_body(dst_hbm, one_hbm, out_hbm, didx, hist, blk, ones_v, hist_sh):
    c = lax.axis_index("c")
    s = lax.axis_index("s")
    wid = c * NS + s
    pltpu.sync_copy(dst_hbm.at[wid], didx)
    pltpu.sync_copy(one_hbm, ones_v)

    ones16 = ones_v[...]
    zeros16 = ones16 - ones16

    def z(i, carry):
        hist[pl.ds(i * 16, 16)] = zeros16
        return carry
    lax.fori_loop(0, NPAD // 16, z, 0)

    kc = jnp.where(c == 0, KF, KS)

    def chunk(j, carry):
        for t_ in range(CH // 16):
            dv = didx[j, pl.ds(t_ * 16, 16)]
            plsc.addupdate_scatter(hist, [dv], ones16)
        return carry
    lax.fori_loop(0, kc, chunk, 0)

    pltpu.sync_copy(hist, hist_sh.at[s])
    plsc.subcore_barrier()
    for t_ in range(NS):
        pltpu.sync_copy(hist_sh.at[t_].at[pl.ds(s * RPS, RPS)], blk.at[t_])

    def red(pp, carry):
        acc = blk[0, pl.ds(pp * 16, 16)]
        for t_ in range(1, NS):
            acc = acc + blk[t_, pl.ds(pp * 16, 16)]
        hist[pl.ds(pp * 16, 16)] = acc
        return carry
    lax.fori_loop(0, RPS // 16, red, 0)
    pltpu.sync_copy(hist.at[pl.ds(0, RPS)],
                    out_hbm.at[c].at[pl.ds(s * RPS, RPS)])


def _sc_deg(dstu):
    f = pl.kernel(
        _sc_deg_body,
        out_type=jax.ShapeDtypeStruct((NC, NPAD), jnp.float32),
        mesh=_mesh(),
        scratch_types=[
            pltpu.VMEM((KMAX, CH), jnp.int32),
            pltpu.VMEM((NPAD,), jnp.float32),
            pltpu.VMEM((NS, RPS), jnp.float32),
            pltpu.VMEM((16,), jnp.float32),
            pltpu.VMEM_SHARED((NS, NPAD), jnp.float32),
        ],
        compiler_params=pltpu.CompilerParams(needs_layout_passes=False),
    )
    return f(dstu, jnp.ones((16,), jnp.float32))


def _sc_pass_body(src_hbm, dst_hbm, feat_hbm, zero_hbm, out_hbm,
                  sidx, didx, rows, acc_sh, sem):
    c = lax.axis_index("c")
    s = lax.axis_index("s")
    wid = c * NS + s
    kc = jnp.where(c == 0, KF, KS)
    pltpu.sync_copy(src_hbm.at[wid], sidx)
    pltpu.sync_copy(dst_hbm.at[wid], didx)

    def zr(t_, carry):
        pltpu.sync_copy(zero_hbm, acc_sh.at[pl.ds(s * RPS + t_ * ZR, ZR)])
        return carry
    lax.fori_loop(0, RPS // ZR, zr, 0)
    plsc.subcore_barrier()

    def chunk(j, carry):
        pltpu.async_copy(feat_hbm.at[sidx.at[j]], rows, sem).wait()
        pltpu.sync_copy(rows, acc_sh.at[didx.at[j]], add=True)
        return carry

    lax.fori_loop(0, kc, chunk, 0)
    plsc.subcore_barrier()
    pltpu.sync_copy(acc_sh.at[pl.ds(s * RPS, RPS)],
                    out_hbm.at[c].at[pl.ds(s * RPS, RPS)])


def _sc_pass(srcp, dstp, feat, zero640):
    f = pl.kernel(
        _sc_pass_body,
        out_type=jax.ShapeDtypeStruct((NC, NPAD, F), jnp.float32),
        mesh=_mesh(),
        scratch_types=[
            pltpu.VMEM((KMAX, CH), jnp.int32),
            pltpu.VMEM((KMAX, CH), jnp.int32),
            pltpu.VMEM((CH, F), jnp.float32),
            pltpu.VMEM_SHARED((NPAD, F), jnp.float32),
            pltpu.SemaphoreType.DMA,
        ],
    )
    return f(srcp, dstp, feat, zero640)


def _tcA_body(d0, d1, x, bt, xs1, dinv, key):
    deg = d0[...] + d1[...]
    din = lax.rsqrt(deg + 1.0)
    dinv[...] = din
    xs1[...] = x[...] * din
    dc = jnp.minimum(deg.astype(jnp.int32), MAXD)
    key[...] = bt[...] * KST + dc


def _tcA(degp, xp, btp):
    row1 = pl.BlockSpec((BN, 1), lambda i: (i, 0))
    rowF = pl.BlockSpec((BN, F), lambda i: (i, 0))
    return pl.pallas_call(
        _tcA_body,
        grid=(GRID,),
        in_specs=[row1, row1, rowF, row1],
        out_specs=[rowF, row1, row1],
        out_shape=[jax.ShapeDtypeStruct((NPAD, F), jnp.float32),
                   jax.ShapeDtypeStruct((NPAD, 1), jnp.float32),
                   jax.ShapeDtypeStruct((NPAD, 1), jnp.int32)],
    )(degp[0].reshape(NPAD, 1), degp[1].reshape(NPAD, 1), xp, btp)


def _tcBC_body(a0, a1, xs, dinv, W, brow, h, xsn):
    p = dinv[...] * (a0[0] + a1[0] + xs[...])
    hv = jnp.dot(p, W[...], preferred_element_type=jnp.float32) + brow[0:1, :]
    h[...] = hv
    xsn[...] = dinv[...] * hv


def _tcBC(acc, xs, dinv, W, b):
    brow = jnp.zeros((8, F), jnp.float32).at[0].set(b)
    row1 = pl.BlockSpec((BN, 1), lambda i: (i, 0))
    rowF = pl.BlockSpec((BN, F), lambda i: (i, 0))
    p0 = pl.BlockSpec((1, BN, F), lambda i: (0, i, 0))
    p1 = pl.BlockSpec((1, BN, F), lambda i: (1, i, 0))
    full = lambda shape: pl.BlockSpec(shape, lambda i: tuple(0 for _ in shape))
    return pl.pallas_call(
        _tcBC_body,
        grid=(GRID,),
        in_specs=[p0, p1, rowF, row1, full((F, F)), full((8, F))],
        out_specs=[rowF, rowF],
        out_shape=[jax.ShapeDtypeStruct((NPAD, F), jnp.float32),
                   jax.ShapeDtypeStruct((NPAD, F), jnp.float32)],
    )(acc, acc, xs, dinv, W, brow)


def _tcD1_body(a0, a1, xs3, ah0, ah1, h2, dinv, bt, ky, P, T, S, cnt):
    i = pl.program_id(0)

    @pl.when(i == 0)
    def _():
        P[...] = jnp.zeros_like(P)
        T[...] = jnp.zeros_like(T)
        S[...] = jnp.zeros_like(S)
        cnt[...] = jnp.zeros_like(cnt)

    polnode = dinv[...] * (a0[0] + a1[0] + xs3[...])
    acch = ah0[0] + ah1[0]
    ob = (bt[...] == lax.broadcasted_iota(jnp.int32, (BN, G), 1)
          ).astype(jnp.float32)
    ok = (ky[...] == lax.broadcasted_iota(jnp.int32, (BN, KW), 1)
          ).astype(jnp.float32)
    dn = (((0,), (0,)), ((), ()))
    P[...] += lax.dot_general(ob, polnode, dn,
                              preferred_element_type=jnp.float32)
    T[...] += lax.dot_general(ok, h2[...], dn,
                              preferred_element_type=jnp.float32)
    S[...] += lax.dot_general(ok, acch, dn,
                              preferred_element_type=jnp.float32)
    cnt[...] += lax.dot_general(ok, jnp.ones((BN, 1), jnp.float32), dn,
                                preferred_element_type=jnp.float32)


def _tcD1(acc3, xs3, acch, h2, dinv, btp, key):
    row1 = pl.BlockSpec((BN, 1), lambda i: (i, 0))
    rowF = pl.BlockSpec((BN, F), lambda i: (i, 0))
    p0 = pl.BlockSpec((1, BN, F), lambda i: (0, i, 0))
    p1 = pl.BlockSpec((1, BN, F), lambda i: (1, i, 0))
    full = lambda shape: pl.BlockSpec(shape, lambda i: tuple(0 for _ in shape))
    return pl.pallas_call(
        _tcD1_body,
        grid=(GRID,),
        in_specs=[p0, p1, rowF, p0, p1, rowF, row1, row1, row1],
        out_specs=[full((G, F)), full((KW, F)), full((KW, F)), full((KW, 1))],
        out_shape=[jax.ShapeDtypeStruct((G, F), jnp.float32),
                   jax.ShapeDtypeStruct((KW, F), jnp.float32),
                   jax.ShapeDtypeStruct((KW, F), jnp.float32),
                   jax.ShapeDtypeStruct((KW, 1), jnp.float32)],
    )(acc3, acc3, xs3, acch, acch, h2, dinv, btp, key)


def _tcD2_body(P, T, S, c, wl, wr, bl, Gm, Wp, bp, pol, val):
    rv = jnp.sum(S[...] * wl[...] + T[...] * wr[...], axis=1, keepdims=True)
    rv = rv + c[...] * bl[...]
    val[...] = jnp.dot(Gm[...], rv, preferred_element_type=jnp.float32)
    counts = jnp.dot(Gm[...], c[...], preferred_element_type=jnp.float32)
    pv = jnp.dot(P[...], Wp[...], preferred_element_type=jnp.float32)
    pol[...] = pv / jnp.maximum(counts, 1.0) + bp[0:1, :]


def _tcD2(P, T, S, c, wlrep, wrrep, blrep, Gmat, W_pol, bp):
    full = lambda shape: pl.BlockSpec(shape, lambda: tuple(0 for _ in shape))
    return pl.pallas_call(
        _tcD2_body,
        in_specs=[full((G, F)), full((KW, F)), full((KW, F)), full((KW, 1)),
                  full((KW, F)), full((KW, F)), full((KW, 1)), full((G, KW)),
                  full((F, G)), full((8, G))],
        out_specs=[full((G, G)), full((G, 1))],
        out_shape=[jax.ShapeDtypeStruct((G, G), jnp.float32),
                   jax.ShapeDtypeStruct((G, 1), jnp.float32)],
    )(P, T, S, c, wlrep, wrrep, blrep, Gmat, W_pol, bp)


def kernel(x, edge_index, batch, W_in, b_in, W_1, b_1,
           Wl_val, bl_val, Wr_val, W_pol, b_pol):
    f32 = jnp.float32
    src = edge_index[0]
    dst = edge_index[1]
    def chunk_rows(ix):
        r = jnp.concatenate([ix, jnp.full((TR * CH - E,), N, jnp.int32)]
                            ).reshape(TR, CH)
        return r

    def skewed(ix):
        r = chunk_rows(ix)
        padrow = jnp.full((NS * (KF + KS) - TR, CH), N, jnp.int32)
        r = jnp.concatenate([r, padrow])
        p0 = r[:NS * KF].reshape(NS, KF, CH)
        p1 = r[NS * KF:].reshape(NS, KS, CH)
        p1 = jnp.concatenate(
            [p1, jnp.full((NS, KMAX - KS, CH), N, jnp.int32)], axis=1)
        return jnp.concatenate([p0, p1])

    srcp = skewed(src)
    dstp = skewed(dst)
    xp = jnp.zeros((NPAD, F), f32).at[:N].set(x)
    btp = jnp.concatenate([batch.astype(jnp.int32),
                           jnp.full((NPAD - N,), G, jnp.int32)]
                          ).reshape(NPAD, 1)
    zero640 = jnp.zeros((ZR, F), f32)

    degp = _sc_deg(dstp)
    xs1, dinv, key = _tcA(degp, xp, btp)
    acc1 = _sc_pass(srcp, dstp, xs1, zero640)
    _h1, xs2 = _tcBC(acc1, xs1, dinv, W_in, b_in)
    acc2 = _sc_pass(srcp, dstp, xs2, zero640)
    h2, xs3 = _tcBC(acc2, xs2, dinv, W_1, b_1)
    acc3 = _sc_pass(srcp, dstp, xs3, zero640)
    acch = _sc_pass(srcp, dstp, h2, zero640)
    P, T, S, c = _tcD1(acc3, xs3, acch, h2, dinv, btp, key)

    wl = Wl_val[:, :, 0]
    wr = Wr_val[:, :, 0]
    bl = bl_val[:, 0]
    wl12 = jnp.zeros((KST, F), f32).at[:MAXD + 1].set(wl)
    wr12 = jnp.zeros((KST, F), f32).at[:MAXD + 1].set(wr)
    bl12 = jnp.zeros((KST,), f32).at[:MAXD + 1].set(bl)
    wlrep = jnp.zeros((KW, F), f32).at[:G * KST].set(jnp.tile(wl12, (G, 1)))
    wrrep = jnp.zeros((KW, F), f32).at[:G * KST].set(jnp.tile(wr12, (G, 1)))
    blrep = jnp.zeros((KW, 1), f32).at[:G * KST, 0].set(jnp.tile(bl12, G))
    col = jnp.arange(KW, dtype=jnp.int32)
    Gmat = (((col[None, :] // KST) == jnp.arange(G, dtype=jnp.int32)[:, None])
            & (col[None, :] < G * KST)).astype(f32)
    bp = jnp.zeros((8, G), f32).at[0].set(b_pol)

    pol, val = _tcD2(P, T, S, c, wlrep, wrrep, blrep, Gmat, W_pol, bp)
    return (pol, val)
